# baseline (speedup 1.0000x reference)
import functools

import jax
import jax.numpy as jnp
from jax import lax
from jax.experimental import pallas as pl
from jax.experimental.pallas import tpu as pltpu
from jax.experimental.pallas import tpu_sc as plsc

N_LUTS = 1024
LUT_DIM = 33
T = 1024
P = 3 * LUT_DIM * LUT_DIM
PB = 33
L = 16


def _build_w_sc(t, nlc, zeros):
    mesh = plsc.VectorSubcoreMesh(core_axis_name="c", subcore_axis_name="s")

    @functools.partial(
        pl.kernel,
        mesh=mesh,
        out_type=jax.ShapeDtypeStruct((N_LUTS,), jnp.float32),
        compiler_params=pltpu.CompilerParams(needs_layout_passes=False),
        scratch_types=[
            pltpu.VMEM((L * N_LUTS,), jnp.float32),
            pltpu.VMEM((T,), jnp.int32),
            pltpu.VMEM((T,), jnp.float32),
            pltpu.VMEM((N_LUTS,), jnp.float32),
        ],
    )
    def k(t_hbm, nlc_hbm, z_hbm, w_hbm, hist_v, t_v, nlc_v, w_v):
        wid = lax.axis_index("s") * 2 + lax.axis_index("c")

        @pl.when(wid == 0)
        def _():
            pltpu.sync_copy(z_hbm, hist_v)
            pltpu.sync_copy(t_hbm, t_v)
            pltpu.sync_copy(nlc_hbm, nlc_v)
            lanes = lax.iota(jnp.int32, L)

            def scat_body(j, carry):
                idx = t_v[pl.ds(j * L, L)]
                val = nlc_v[pl.ds(j * L, L)]
                plsc.addupdate_scatter(hist_v, [lanes * N_LUTS + idx], val)
                return carry

            lax.fori_loop(0, T // L, scat_body, 0)

            def red_body(j, carry):
                acc = hist_v[pl.ds(j * L, L)]
                for r in range(1, L):
                    acc = acc + hist_v[pl.ds(r * N_LUTS + j * L, L)]
                w_v[pl.ds(j * L, L)] = acc
                return carry

            lax.fori_loop(0, N_LUTS // L, red_body, 0)
            pltpu.sync_copy(w_v, w_hbm)

    return k(t, nlc, zeros)


def _matvec_body(w_ref, blk0_ref, blk1_ref, blk2_ref, out_ref):
    out_ref[0] = jnp.sum(blk0_ref[...] * w_ref[...], axis=-1)
    out_ref[1] = jnp.sum(blk1_ref[...] * w_ref[...], axis=-1)
    out_ref[2] = jnp.sum(blk2_ref[...] * w_ref[...], axis=-1)


def kernel(t, nlc, lut):
    w = jnp.zeros((N_LUTS,), jnp.float32).at[t].add(nlc)
    lut3 = jnp.transpose(lut, (1, 2, 3, 4, 0)).reshape(P, LUT_DIM, N_LUTS)
    out2 = pl.pallas_call(
        _matvec_body,
        grid=(P // PB // 3,),
        in_specs=[
            pl.BlockSpec((N_LUTS,), lambda i: (0,)),
            pl.BlockSpec((PB, LUT_DIM, N_LUTS), lambda i: (3 * i, 0, 0)),
            pl.BlockSpec((PB, LUT_DIM, N_LUTS), lambda i: (3 * i + 1, 0, 0)),
            pl.BlockSpec((PB, LUT_DIM, N_LUTS), lambda i: (3 * i + 2, 0, 0)),
        ],
        out_specs=pl.BlockSpec((3, PB, LUT_DIM), lambda i: (i, 0, 0)),
        out_shape=jax.ShapeDtypeStruct((P // PB, PB, LUT_DIM), jnp.float32),
    )(w, lut3, lut3, lut3)
    return out2.reshape(3, LUT_DIM, LUT_DIM, LUT_DIM)


# speedup vs baseline: 1.1717x; 1.1717x over previous
import functools

import jax
import jax.numpy as jnp
from jax import lax
from jax.experimental import pallas as pl
from jax.experimental.pallas import tpu as pltpu
from jax.experimental.pallas import tpu_sc as plsc

N_LUTS = 1024
LUT_DIM = 33
T = 1024
P = 3 * LUT_DIM * LUT_DIM
PB = 33
L = 16


def _build_w_sc(t, nlc, zeros):
    mesh = plsc.VectorSubcoreMesh(core_axis_name="c", subcore_axis_name="s")

    @functools.partial(
        pl.kernel,
        mesh=mesh,
        out_type=jax.ShapeDtypeStruct((N_LUTS,), jnp.float32),
        compiler_params=pltpu.CompilerParams(needs_layout_passes=False),
        scratch_types=[
            pltpu.VMEM((L * N_LUTS,), jnp.float32),
            pltpu.VMEM((T,), jnp.int32),
            pltpu.VMEM((T,), jnp.float32),
            pltpu.VMEM((N_LUTS,), jnp.float32),
            pltpu.SemaphoreType.DMA,
            pltpu.SemaphoreType.DMA,
            pltpu.SemaphoreType.DMA,
        ],
    )
    def k(t_hbm, nlc_hbm, z_hbm, w_hbm, hist_v, t_v, nlc_v, w_v,
          sem0, sem1, sem2):
        wid = lax.axis_index("s") * 2 + lax.axis_index("c")

        @pl.when(wid == 0)
        def _():
            cz = pltpu.make_async_copy(z_hbm, hist_v, sem0)
            ct = pltpu.make_async_copy(t_hbm, t_v, sem1)
            cn = pltpu.make_async_copy(nlc_hbm, nlc_v, sem2)
            cz.start()
            ct.start()
            cn.start()
            cz.wait()
            ct.wait()
            cn.wait()
            lanes = lax.iota(jnp.int32, L)

            def scat_body(j, carry):
                for u in range(2):
                    idx = t_v[pl.ds((2 * j + u) * L, L)]
                    val = nlc_v[pl.ds((2 * j + u) * L, L)]
                    plsc.addupdate_scatter(hist_v, [lanes * N_LUTS + idx], val)
                return carry

            lax.fori_loop(0, T // L // 2, scat_body, 0)

            def red_body(j, carry):
                for u in range(2):
                    jj = 2 * j + u
                    acc = hist_v[pl.ds(jj * L, L)]
                    for r in range(1, L):
                        acc = acc + hist_v[pl.ds(r * N_LUTS + jj * L, L)]
                    w_v[pl.ds(jj * L, L)] = acc
                return carry

            lax.fori_loop(0, N_LUTS // L // 2, red_body, 0)
            pltpu.sync_copy(w_v, w_hbm)

    return k(t, nlc, zeros)


def _matvec_body(w_ref, blk0_ref, blk1_ref, blk2_ref, out_ref):
    out_ref[0] = jnp.sum(blk0_ref[...] * w_ref[...], axis=-1)
    out_ref[1] = jnp.sum(blk1_ref[...] * w_ref[...], axis=-1)
    out_ref[2] = jnp.sum(blk2_ref[...] * w_ref[...], axis=-1)


def kernel(t, nlc, lut):
    zeros = jnp.zeros((L * N_LUTS,), jnp.float32)
    w = _build_w_sc(t.astype(jnp.int32), nlc, zeros)
    lut3 = jnp.transpose(lut, (1, 2, 3, 4, 0)).reshape(P, LUT_DIM, N_LUTS)
    out2 = pl.pallas_call(
        _matvec_body,
        grid=(P // PB // 3,),
        in_specs=[
            pl.BlockSpec((N_LUTS,), lambda i: (0,)),
            pl.BlockSpec((PB, LUT_DIM, N_LUTS), lambda i: (3 * i, 0, 0)),
            pl.BlockSpec((PB, LUT_DIM, N_LUTS), lambda i: (3 * i + 1, 0, 0)),
            pl.BlockSpec((PB, LUT_DIM, N_LUTS), lambda i: (3 * i + 2, 0, 0)),
        ],
        out_specs=pl.BlockSpec((3, PB, LUT_DIM), lambda i: (i, 0, 0)),
        out_shape=jax.ShapeDtypeStruct((P // PB, PB, LUT_DIM), jnp.float32),
    )(w, lut3, lut3, lut3)
    return out2.reshape(3, LUT_DIM, LUT_DIM, LUT_DIM)
